# baseline (speedup 1.0000x reference)
import functools

import jax
import jax.numpy as jnp
from jax import lax
from jax.experimental import pallas as pl
from jax.experimental.pallas import tpu as pltpu
from jax.experimental.pallas import tpu_sc as plsc

B = 16384
F = 26
VEC = 32
OUT = 16
N = B * F

NC = 2
NS = 16
NW = NC * NS
PER_W = N // NW
CHUNK = 1024
ROWS_PER_DMA = 128
DMAS_PER_CHUNK = CHUNK // ROWS_PER_DMA
STEPS = PER_W // CHUNK

TOK_BLK = 13312
GRID = N // TOK_BLK


V = 1000000
FMT_W = 1000
FMT_CHUNKS = V // FMT_W


def _sc_format(tableT):
    mesh = plsc.VectorSubcoreMesh(core_axis_name="c", subcore_axis_name="s")

    @functools.partial(
        pl.kernel,
        out_type=jax.ShapeDtypeStruct((V, VEC), jnp.float32),
        mesh=mesh,
        scratch_types=[
            pltpu.VMEM((VEC, FMT_W), jnp.float32),
            pltpu.VMEM((FMT_W, VEC), jnp.float32),
        ],
        compiler_params=pltpu.CompilerParams(use_tc_tiling_on_sc=False,
                                             needs_layout_passes=False),
    )
    def fmt_kernel(tt_hbm, out_hbm, panel, outp):
        wid = lax.axis_index("s") * NC + lax.axis_index("c")
        iota_lo = lax.broadcasted_iota(jnp.int32, (16,), 0)
        iota_hi = iota_lo + 16

        def chunk(ci, _):
            c = ci * NW + wid

            @pl.when(c < FMT_CHUNKS)
            def _():
                base = c * FMT_W
                pltpu.sync_copy(tt_hbm.at[:, pl.ds(base, FMT_W)], panel)

                def rows(rb, _):
                    for u in range(8):
                        r = rb * 8 + u
                        cols = jnp.full((16,), r, jnp.int32)
                        outp[r, pl.ds(0, 16)] = plsc.load_gather(
                            panel, [iota_lo, cols])
                        outp[r, pl.ds(16, 16)] = plsc.load_gather(
                            panel, [iota_hi, cols])
                    return ()

                lax.fori_loop(0, FMT_W // 8, rows, (), unroll=False)
                pltpu.sync_copy(outp, out_hbm.at[pl.ds(base, FMT_W)])

            return ()

        lax.fori_loop(0, FMT_CHUNKS // NW + 1, chunk, (), unroll=False)

    return fmt_kernel(tableT)


def _sc_gather(table, idx2d):
    mesh = plsc.VectorSubcoreMesh(core_axis_name="c", subcore_axis_name="s")

    @functools.partial(
        pl.kernel,
        out_type=(
            jax.ShapeDtypeStruct((N, VEC), jnp.float32),
            jax.ShapeDtypeStruct((N // ROWS_PER_DMA, ROWS_PER_DMA),
                                 jnp.float32),
        ),
        mesh=mesh,
        scratch_types=[
            pltpu.VMEM((DMAS_PER_CHUNK, ROWS_PER_DMA), jnp.int32),
            pltpu.VMEM((DMAS_PER_CHUNK, ROWS_PER_DMA), jnp.float32),
            pltpu.VMEM((CHUNK, VEC), jnp.float32),
            pltpu.SemaphoreType.DMA,
        ],
        compiler_params=pltpu.CompilerParams(use_tc_tiling_on_sc=False),
    )
    def gather_kernel(table_hbm, idx_hbm, out_hbm, msk_hbm, idx_v, msk_v,
                      rows_v, sem):
        wid = lax.axis_index("s") * NC + lax.axis_index("c")

        def step(s, _):
            row_base = wid * (PER_W // ROWS_PER_DMA) + s * DMAS_PER_CHUNK
            base = wid * PER_W + s * CHUNK
            pltpu.sync_copy(idx_hbm.at[pl.ds(row_base, DMAS_PER_CHUNK)], idx_v)
            for r in range(DMAS_PER_CHUNK):
                for c in range(ROWS_PER_DMA // 16):
                    v = idx_v[r, pl.ds(c * 16, 16)]
                    msk_v[r, pl.ds(c * 16, 16)] = jnp.where(
                        v < 1, 1.0, 0.0).astype(jnp.float32)
                    idx_v[r, pl.ds(c * 16, 16)] = jnp.maximum(v - 1, 0)
            handles = []
            for j in range(DMAS_PER_CHUNK):
                handles.append(pltpu.async_copy(
                    table_hbm.at[idx_v.at[j]],
                    rows_v.at[pl.ds(j * ROWS_PER_DMA, ROWS_PER_DMA)],
                    sem,
                ))
            for h in handles:
                h.wait()
            pltpu.sync_copy(rows_v, out_hbm.at[pl.ds(base, CHUNK)])
            pltpu.sync_copy(msk_v, msk_hbm.at[pl.ds(row_base, DMAS_PER_CHUNK)])
            return ()

        lax.fori_loop(0, STEPS, step, (), unroll=False)

    return gather_kernel(table, idx2d)


def _mlp_body(x_ref, m_ref, u_ref, w1_ref, b1_ref, w2_ref, b2_ref,
              w3_ref, b3_ref, o_ref):
    x = x_ref[...]
    h = jnp.dot(x.astype(jnp.bfloat16), w1_ref[...],
                preferred_element_type=jnp.float32) + b1_ref[...]
    h = jnp.maximum(h, 0.0)
    h = jnp.dot(h.astype(jnp.bfloat16), w2_ref[...],
                preferred_element_type=jnp.float32) + b2_ref[...]
    h = jnp.maximum(h, 0.0)
    o = jnp.dot(h.astype(jnp.bfloat16), w3_ref[...],
                preferred_element_type=jnp.float32) + b3_ref[...]
    nr = TOK_BLK // 128
    m = m_ref[...]
    mb = lax.broadcast_in_dim(m.reshape(nr, 1, 128), (nr, 32, 128),
                              (0, 1, 2))
    lm = mb.reshape(TOK_BLK // 4, 128)
    row = lax.broadcasted_iota(jnp.int32, (TOK_BLK // 4, 128), 0)
    lane = lax.broadcasted_iota(jnp.int32, (TOK_BLK // 4, 128), 1)
    lane64 = lax.broadcasted_iota(jnp.int32, (1, 64), 1)
    mrep = jnp.zeros((TOK_BLK // 4, 64), jnp.float32)
    for b in range(4):
        sel = lane == (4 * (row & 31) + b)
        rs = jnp.sum(jnp.where(sel, lm, 0.0), axis=1, keepdims=True)
        band = jnp.where((lane64 >= 16 * b) & (lane64 < 16 * (b + 1)),
                         1.0, 0.0)
        mrep = mrep + rs * band
    o = jnp.where(mrep > 0.5, u_ref[...], o)
    o_ref[...] = o


def _tc_mlp(x4, msk, u64, W1b, b1b, W2b, b2b, W3b, b3b):
    return pl.pallas_call(
        _mlp_body,
        grid=(GRID,),
        in_specs=[
            pl.BlockSpec((TOK_BLK // 4, 128), lambda i: (i, 0)),
            pl.BlockSpec((TOK_BLK // 128, 128), lambda i: (i, 0)),
            pl.BlockSpec((1, 64), lambda i: (0, 0)),
            pl.BlockSpec((128, 128), lambda i: (0, 0)),
            pl.BlockSpec((1, 128), lambda i: (0, 0)),
            pl.BlockSpec((128, 128), lambda i: (0, 0)),
            pl.BlockSpec((1, 128), lambda i: (0, 0)),
            pl.BlockSpec((128, 64), lambda i: (0, 0)),
            pl.BlockSpec((1, 64), lambda i: (0, 0)),
        ],
        out_specs=pl.BlockSpec((TOK_BLK // 4, 64), lambda i: (i, 0)),
        out_shape=jax.ShapeDtypeStruct((N // 4, 64), jnp.float32),
    )(x4, msk, u64, W1b, b1b, W2b, b2b, W3b, b3b)


def _block_diag4(W, rows, cols):
    h, w = W.shape
    out = jnp.zeros((4, rows, 4, cols), jnp.float32)
    pad = jnp.zeros((rows, cols), jnp.float32).at[:h, :w].set(W)
    for g in range(4):
        out = out.at[g, :, g, :].set(pad)
    return out.reshape(4 * rows, 4 * cols)


def kernel(indices, unknown, table, W1, b1, W2, b2, W3, b3):
    idx_flat = indices.reshape(N)
    idx2d = idx_flat.reshape(N // ROWS_PER_DMA, ROWS_PER_DMA)
    table_lin = _sc_format(table.T)
    gathered, msk = _sc_gather(table_lin, idx2d)
    x4 = gathered.reshape(N // 4, 128)

    h1, h2 = W1.shape[1], W2.shape[1]
    W1b = _block_diag4(W1, VEC, VEC).astype(jnp.bfloat16)
    W2b = _block_diag4(W2, VEC, VEC).astype(jnp.bfloat16)
    W3b = _block_diag4(W3, VEC, OUT).astype(jnp.bfloat16)
    b1b = jnp.tile(jnp.zeros((VEC,), jnp.float32).at[:h1].set(b1),
                   4).reshape(1, 128)
    b2b = jnp.tile(jnp.zeros((VEC,), jnp.float32).at[:h2].set(b2),
                   4).reshape(1, 128)
    b3b = jnp.tile(b3, 4).reshape(1, 64)
    u64 = jnp.tile(unknown.reshape(OUT), 4).reshape(1, 64)

    out4 = _tc_mlp(x4, msk, u64, W1b, b1b, W2b, b2b, W3b, b3b)
    return out4.reshape(B, F, OUT)


# speedup vs baseline: 3.7525x; 3.7525x over previous
import functools

import jax
import jax.numpy as jnp
from jax import lax
from jax.experimental import pallas as pl
from jax.experimental.pallas import tpu as pltpu
from jax.experimental.pallas import tpu_sc as plsc

B = 16384
F = 26
VEC = 32
OUT = 16
N = B * F

NC = 2
NS = 16
NW = NC * NS
PER_W = N // NW
CHUNK = 1024
ROWS_PER_DMA = 128
DMAS_PER_CHUNK = CHUNK // ROWS_PER_DMA
STEPS = PER_W // CHUNK

TOK_BLK = 13312
GRID = N // TOK_BLK


def _sc_gather(table, idx2d):
    mesh = plsc.VectorSubcoreMesh(core_axis_name="c", subcore_axis_name="s")

    @functools.partial(
        pl.kernel,
        out_type=(
            jax.ShapeDtypeStruct((N, VEC), jnp.float32),
            jax.ShapeDtypeStruct((N // ROWS_PER_DMA, ROWS_PER_DMA),
                                 jnp.float32),
        ),
        mesh=mesh,
        scratch_types=[
            pltpu.VMEM((DMAS_PER_CHUNK, ROWS_PER_DMA), jnp.int32),
            pltpu.VMEM((DMAS_PER_CHUNK, ROWS_PER_DMA), jnp.float32),
            pltpu.VMEM((CHUNK, VEC), jnp.float32),
            pltpu.SemaphoreType.DMA,
        ],
        compiler_params=pltpu.CompilerParams(use_tc_tiling_on_sc=False),
    )
    def gather_kernel(table_hbm, idx_hbm, out_hbm, msk_hbm, idx_v, msk_v,
                      rows_v, sem):
        wid = lax.axis_index("s") * NC + lax.axis_index("c")

        def step(s, _):
            row_base = wid * (PER_W // ROWS_PER_DMA) + s * DMAS_PER_CHUNK
            base = wid * PER_W + s * CHUNK
            pltpu.sync_copy(idx_hbm.at[pl.ds(row_base, DMAS_PER_CHUNK)], idx_v)
            for r in range(DMAS_PER_CHUNK):
                for c in range(ROWS_PER_DMA // 16):
                    v = idx_v[r, pl.ds(c * 16, 16)]
                    msk_v[r, pl.ds(c * 16, 16)] = jnp.where(
                        v < 1, 1.0, 0.0).astype(jnp.float32)
                    idx_v[r, pl.ds(c * 16, 16)] = jnp.maximum(v - 1, 0)
            handles = []
            for j in range(DMAS_PER_CHUNK):
                handles.append(pltpu.async_copy(
                    table_hbm.at[idx_v.at[j]],
                    rows_v.at[pl.ds(j * ROWS_PER_DMA, ROWS_PER_DMA)],
                    sem,
                ))
            for h in handles:
                h.wait()
            pltpu.sync_copy(rows_v, out_hbm.at[pl.ds(base, CHUNK)])
            pltpu.sync_copy(msk_v, msk_hbm.at[pl.ds(row_base, DMAS_PER_CHUNK)])
            return ()

        lax.fori_loop(0, STEPS, step, (), unroll=False)

    return gather_kernel(table, idx2d)


def _mlp_body(x_ref, m_ref, u_ref, w1_ref, b1_ref, w2_ref, b2_ref,
              w3_ref, b3_ref, o_ref):
    x = x_ref[...]
    h = jnp.dot(x.astype(jnp.bfloat16), w1_ref[...],
                preferred_element_type=jnp.float32) + b1_ref[...]
    h = jnp.maximum(h, 0.0)
    h = jnp.dot(h.astype(jnp.bfloat16), w2_ref[...],
                preferred_element_type=jnp.float32) + b2_ref[...]
    h = jnp.maximum(h, 0.0)
    o = jnp.dot(h.astype(jnp.bfloat16), w3_ref[...],
                preferred_element_type=jnp.float32) + b3_ref[...]
    nr = TOK_BLK // 128
    m = m_ref[...]
    mb = lax.broadcast_in_dim(m.reshape(nr, 1, 128), (nr, 32, 128),
                              (0, 1, 2))
    lm = mb.reshape(TOK_BLK // 4, 128)
    row = lax.broadcasted_iota(jnp.int32, (TOK_BLK // 4, 128), 0)
    lane = lax.broadcasted_iota(jnp.int32, (TOK_BLK // 4, 128), 1)
    lane64 = lax.broadcasted_iota(jnp.int32, (1, 64), 1)
    mrep = jnp.zeros((TOK_BLK // 4, 64), jnp.float32)
    for b in range(4):
        sel = lane == (4 * (row & 31) + b)
        rs = jnp.sum(jnp.where(sel, lm, 0.0), axis=1, keepdims=True)
        band = jnp.where((lane64 >= 16 * b) & (lane64 < 16 * (b + 1)),
                         1.0, 0.0)
        mrep = mrep + rs * band
    o = jnp.where(mrep > 0.5, u_ref[...], o)
    o_ref[...] = o


def _tc_mlp(x4, msk, u64, W1b, b1b, W2b, b2b, W3b, b3b):
    return pl.pallas_call(
        _mlp_body,
        grid=(GRID,),
        in_specs=[
            pl.BlockSpec((TOK_BLK // 4, 128), lambda i: (i, 0)),
            pl.BlockSpec((TOK_BLK // 128, 128), lambda i: (i, 0)),
            pl.BlockSpec((1, 64), lambda i: (0, 0)),
            pl.BlockSpec((128, 128), lambda i: (0, 0)),
            pl.BlockSpec((1, 128), lambda i: (0, 0)),
            pl.BlockSpec((128, 128), lambda i: (0, 0)),
            pl.BlockSpec((1, 128), lambda i: (0, 0)),
            pl.BlockSpec((128, 64), lambda i: (0, 0)),
            pl.BlockSpec((1, 64), lambda i: (0, 0)),
        ],
        out_specs=pl.BlockSpec((TOK_BLK // 4, 64), lambda i: (i, 0)),
        out_shape=jax.ShapeDtypeStruct((N // 4, 64), jnp.float32),
    )(x4, msk, u64, W1b, b1b, W2b, b2b, W3b, b3b)


def _block_diag4(W, rows, cols):
    h, w = W.shape
    out = jnp.zeros((4, rows, 4, cols), jnp.float32)
    pad = jnp.zeros((rows, cols), jnp.float32).at[:h, :w].set(W)
    for g in range(4):
        out = out.at[g, :, g, :].set(pad)
    return out.reshape(4 * rows, 4 * cols)


def kernel(indices, unknown, table, W1, b1, W2, b2, W3, b3):
    idx_flat = indices.reshape(N)
    idx2d = idx_flat.reshape(N // ROWS_PER_DMA, ROWS_PER_DMA)
    gathered, msk = _sc_gather(table, idx2d)
    x4 = gathered.reshape(N // 4, 128)

    h1, h2 = W1.shape[1], W2.shape[1]
    W1b = _block_diag4(W1, VEC, VEC).astype(jnp.bfloat16)
    W2b = _block_diag4(W2, VEC, VEC).astype(jnp.bfloat16)
    W3b = _block_diag4(W3, VEC, OUT).astype(jnp.bfloat16)
    b1b = jnp.tile(jnp.zeros((VEC,), jnp.float32).at[:h1].set(b1),
                   4).reshape(1, 128)
    b2b = jnp.tile(jnp.zeros((VEC,), jnp.float32).at[:h2].set(b2),
                   4).reshape(1, 128)
    b3b = jnp.tile(b3, 4).reshape(1, 64)
    u64 = jnp.tile(unknown.reshape(OUT), 4).reshape(1, 64)

    out4 = _tc_mlp(x4, msk, u64, W1b, b1b, W2b, b2b, W3b, b3b)
    return out4.reshape(B, F, OUT)
